# baseline (speedup 1.0000x reference)
import functools

import jax
import jax.numpy as jnp
from jax import lax
from jax.experimental import pallas as pl
from jax.experimental.pallas import tpu as pltpu
from jax.experimental.pallas import tpu_sc as plsc

T_STATIC = 32


def _gather_rows_sc(spatial_table, temporal_table, spatial_idx, temporal_idx):
    ns, d = spatial_table.shape
    nt = temporal_idx.shape[0]
    info = plsc.get_sparse_core_info()
    nw = info.num_cores * info.num_subcores
    rows_s = ns // nw
    t_chunk = 8
    nt_workers = nt // t_chunk
    mesh = plsc.VectorSubcoreMesh(core_axis_name="c", subcore_axis_name="s")

    @functools.partial(
        pl.kernel,
        mesh=mesh,
        out_type=(
            jax.ShapeDtypeStruct((ns, d), jnp.float32),
            jax.ShapeDtypeStruct((nt, d), jnp.float32),
        ),
        scratch_types=[
            pltpu.VMEM((rows_s,), jnp.int32),
            pltpu.VMEM((rows_s, d), jnp.float32),
            pltpu.VMEM((t_chunk,), jnp.int32),
            pltpu.VMEM((t_chunk, d), jnp.float32),
            pltpu.SemaphoreType.DMA,
            pltpu.SemaphoreType.DMA,
        ],
    )
    def k(st_hbm, tt_hbm, sidx_hbm, tidx_hbm, out_s, out_t,
          sidx_v, srows_v, tidx_v, trows_v, sem_s, sem_t):
        wid = lax.axis_index("s") * info.num_cores + lax.axis_index("c")
        base = wid * rows_s
        pltpu.sync_copy(sidx_hbm.at[pl.ds(base, rows_s)], sidx_v)
        s_gather = pltpu.async_copy(st_hbm.at[sidx_v], srows_v, sem_s)

        @pl.when(wid < nt_workers)
        def _temporal():
            tbase = wid * t_chunk
            pltpu.sync_copy(tidx_hbm.at[pl.ds(tbase, t_chunk)], tidx_v)
            pltpu.async_copy(tt_hbm.at[tidx_v], trows_v, sem_t).wait()
            pltpu.sync_copy(trows_v, out_t.at[pl.ds(tbase, t_chunk)])

        s_gather.wait()
        pltpu.sync_copy(srows_v, out_s.at[pl.ds(base, rows_s)])

    return k(spatial_table, temporal_table, spatial_idx, temporal_idx)


def kernel(B, T, Ns, spatial_table, temporal_table):
    ns_static = spatial_table.shape[0]
    nt_rows = temporal_table.shape[0]
    s_off = jnp.asarray(Ns, jnp.int32) - ns_static
    t_off = jnp.asarray(T, jnp.int32) - T_STATIC
    spatial_idx = jnp.clip(
        jnp.arange(ns_static, dtype=jnp.int32) + s_off, 0, ns_static - 1)
    temporal_idx = jnp.clip(
        jnp.arange(T_STATIC, dtype=jnp.int32) + t_off, 0, nt_rows - 1)
    spatial_pe, temporal_pe = _gather_rows_sc(
        spatial_table, temporal_table, spatial_idx, temporal_idx)
    return (spatial_pe[None, None, :, :], temporal_pe[None, :, None, :])
